# baseline (speedup 1.0000x reference)
import jax
import jax.numpy as jnp
from jax import lax
from jax.experimental import pallas as pl
from jax.experimental.pallas import tpu as pltpu
from jax.experimental.pallas import tpu_sc as plsc

NC = 2
NS = 16
NW = NC * NS
L = 16
B = 16384
D = 64
W = 128
BPW = B // NW
C = 128
NSL = D // L


def _sc_body(h_hbm, r_hbm, t_hbm, e1_hbm, e2_hbm, r1_hbm, r2_hbm, out_hbm,
             hidx_v, ridx_v, tidx_v, hph_v, rph_v, tph_v,
             e1h_v, e2h_v, r1_v, r2_v, e1t_v, e2t_v, out_v, sem):
    cid = lax.axis_index("c")
    sid = lax.axis_index("s")
    wid = sid * NC + cid
    base = wid * BPW
    lane = lax.iota(jnp.int32, L)

    def chunk(j, carry):
        off = base + j * C
        pltpu.sync_copy(h_hbm.at[pl.ds(off, C)], hidx_v)
        pltpu.sync_copy(r_hbm.at[pl.ds(off, C)], ridx_v)
        pltpu.sync_copy(t_hbm.at[pl.ds(off, C)], tidx_v)
        for q in range(C // L):
            sl = pl.ds(q * L, L)
            hph_v[sl] = lax.shift_right_logical(hidx_v[sl], 1)
            rph_v[sl] = lax.shift_right_logical(ridx_v[sl], 1)
            tph_v[sl] = lax.shift_right_logical(tidx_v[sl], 1)
        cps = [
            pltpu.async_copy(e1_hbm.at[hph_v], e1h_v, sem),
            pltpu.async_copy(e2_hbm.at[hph_v], e2h_v, sem),
            pltpu.async_copy(r1_hbm.at[rph_v], r1_v, sem),
            pltpu.async_copy(r2_hbm.at[rph_v], r2_v, sem),
            pltpu.async_copy(e1_hbm.at[tph_v], e1t_v, sem),
            pltpu.async_copy(e2_hbm.at[tph_v], e2t_v, sem),
        ]
        for cp in cps:
            cp.wait()

        def group(g, carry2):
            gsl = pl.ds(g * L, L)
            hid16 = hidx_v[gsl]
            rid16 = ridx_v[gsl]
            tid16 = tidx_v[gsl]
            res = jnp.zeros((L,), jnp.float32)
            for k in range(L):
                i = g * L + k
                hoff = (hid16[k] & 1) * D
                roff = (rid16[k] & 1) * D
                toff = (tid16[k] & 1) * D
                acc = jnp.zeros((L,), jnp.float32)
                for s in range(NSL):
                    hsl = pl.ds(hoff + s * L, L)
                    rsl = pl.ds(roff + s * L, L)
                    tsl = pl.ds(toff + s * L, L)
                    acc = (acc
                           + e1h_v[i, hsl] * r1_v[i, rsl] * e2t_v[i, tsl]
                           + e2h_v[i, hsl] * r2_v[i, rsl] * e1t_v[i, tsl])
                res = jnp.where(lane == k, jnp.sum(acc), res)
            out_v[pl.ds(g * L, L)] = res * 0.5
            return carry2

        lax.fori_loop(0, C // L, group, 0)
        pltpu.sync_copy(out_v, out_hbm.at[pl.ds(off, C)])
        return carry

    lax.fori_loop(0, BPW // C, chunk, 0)


def kernel(h_idx, r_idx, t_idx, E1, E2, R1, R2):
    h = h_idx.astype(jnp.int32)
    r = r_idx.astype(jnp.int32)
    t = t_idx.astype(jnp.int32)
    mesh = plsc.VectorSubcoreMesh(core_axis_name="c", subcore_axis_name="s")
    fn = pl.kernel(
        _sc_body,
        mesh=mesh,
        compiler_params=pltpu.CompilerParams(needs_layout_passes=False),
        out_type=jax.ShapeDtypeStruct((B,), jnp.float32),
        scratch_types=[
            pltpu.VMEM((C,), jnp.int32),
            pltpu.VMEM((C,), jnp.int32),
            pltpu.VMEM((C,), jnp.int32),
            pltpu.VMEM((C,), jnp.int32),
            pltpu.VMEM((C,), jnp.int32),
            pltpu.VMEM((C,), jnp.int32),
            pltpu.VMEM((C, W), jnp.float32),
            pltpu.VMEM((C, W), jnp.float32),
            pltpu.VMEM((C, W), jnp.float32),
            pltpu.VMEM((C, W), jnp.float32),
            pltpu.VMEM((C, W), jnp.float32),
            pltpu.VMEM((C, W), jnp.float32),
            pltpu.VMEM((C,), jnp.float32),
            pltpu.SemaphoreType.DMA,
        ],
    )
    return fn(h, r, t,
              E1.reshape(-1, W), E2.reshape(-1, W),
              R1.reshape(-1, W), R2.reshape(-1, W))


# speedup vs baseline: 1.2704x; 1.2704x over previous
import jax
import jax.numpy as jnp
from jax import lax
from jax.experimental import pallas as pl
from jax.experimental.pallas import tpu as pltpu
from jax.experimental.pallas import tpu_sc as plsc

NC = 2
NS = 16
NW = NC * NS
L = 16
B = 16384
D = 64
BPW = B // NW
C = 16
NSL = D // L


def _sc_body(h_hbm, r_hbm, t_hbm, e1_hbm, e2_hbm, r1_hbm, r2_hbm, out_hbm,
             hidx_v, ridx_v, tidx_v,
             e1h_v, e2h_v, r1_v, r2_v, e1t_v, e2t_v, out_v, sem):
    cid = lax.axis_index("c")
    sid = lax.axis_index("s")
    wid = sid * NC + cid
    base = wid * BPW
    lane = lax.iota(jnp.int32, L)

    def chunk(j, carry):
        off = base + j * C
        pltpu.sync_copy(h_hbm.at[pl.ds(off, C)], hidx_v)
        pltpu.sync_copy(r_hbm.at[pl.ds(off, C)], ridx_v)
        pltpu.sync_copy(t_hbm.at[pl.ds(off, C)], tidx_v)
        hid = hidx_v[pl.ds(0, L)]
        rid = ridx_v[pl.ds(0, L)]
        tid = tidx_v[pl.ds(0, L)]
        for k in range(C):
            hrow = pl.multiple_of((hid[k] >> 3) * 8, 8)
            rrow = pl.multiple_of((rid[k] >> 3) * 8, 8)
            trow = pl.multiple_of((tid[k] >> 3) * 8, 8)
            pltpu.async_copy(
                e1_hbm.at[pl.ds(hrow, 8), :], e1h_v.at[k], sem)
            pltpu.async_copy(
                e2_hbm.at[pl.ds(hrow, 8), :], e2h_v.at[k], sem)
            pltpu.async_copy(
                r1_hbm.at[pl.ds(rrow, 8), :], r1_v.at[k], sem)
            pltpu.async_copy(
                r2_hbm.at[pl.ds(rrow, 8), :], r2_v.at[k], sem)
            pltpu.async_copy(
                e1_hbm.at[pl.ds(trow, 8), :], e1t_v.at[k], sem)
            pltpu.async_copy(
                e2_hbm.at[pl.ds(trow, 8), :], e2t_v.at[k], sem)
        for buf in (e1h_v, e2h_v, r1_v, r2_v, e1t_v, e2t_v):
            pltpu.make_async_copy(
                e1_hbm.at[pl.ds(0, 8 * C), :], buf, sem).wait()

        res = jnp.zeros((L,), jnp.float32)
        for k in range(C):
            hs = hid[k] & 7
            rs = rid[k] & 7
            ts = tid[k] & 7
            acc = jnp.zeros((L,), jnp.float32)
            for s in range(NSL):
                sl = pl.ds(s * L, L)
                acc = (acc
                       + e1h_v[k, hs, sl] * r1_v[k, rs, sl] * e2t_v[k, ts, sl]
                       + e2h_v[k, hs, sl] * r2_v[k, rs, sl] * e1t_v[k, ts, sl])
            res = jnp.where(lane == k, jnp.sum(acc), res)
        out_v[...] = res * 0.5
        pltpu.sync_copy(out_v, out_hbm.at[pl.ds(off, C)])
        return carry

    lax.fori_loop(0, BPW // C, chunk, 0)


def kernel(h_idx, r_idx, t_idx, E1, E2, R1, R2):
    h = h_idx.astype(jnp.int32)
    r = r_idx.astype(jnp.int32)
    t = t_idx.astype(jnp.int32)
    mesh = plsc.VectorSubcoreMesh(core_axis_name="c", subcore_axis_name="s")
    fn = pl.kernel(
        _sc_body,
        mesh=mesh,
        compiler_params=pltpu.CompilerParams(needs_layout_passes=False),
        out_type=jax.ShapeDtypeStruct((B,), jnp.float32),
        scratch_types=[
            pltpu.VMEM((C,), jnp.int32),
            pltpu.VMEM((C,), jnp.int32),
            pltpu.VMEM((C,), jnp.int32),
            pltpu.VMEM((C, 8, D), jnp.float32),
            pltpu.VMEM((C, 8, D), jnp.float32),
            pltpu.VMEM((C, 8, D), jnp.float32),
            pltpu.VMEM((C, 8, D), jnp.float32),
            pltpu.VMEM((C, 8, D), jnp.float32),
            pltpu.VMEM((C, 8, D), jnp.float32),
            pltpu.VMEM((C,), jnp.float32),
            pltpu.SemaphoreType.DMA,
        ],
    )
    return fn(h, r, t, E1, E2, R1, R2)
